# Initial kernel scaffold; baseline (speedup 1.0000x reference)
import jax
import jax.numpy as jnp
from jax.experimental import pallas as pl


def kernel(code_ids, seg_ids, word_table, seg_table, ln_gamma, ln_beta):
    raise NotImplementedError("write your pallas kernel here")


# speedup vs baseline: 3.9070x; 3.9070x over previous
import jax
import jax.numpy as jnp
from jax import lax
from jax.experimental import pallas as pl
from jax.experimental.pallas import tpu as pltpu
from jax.experimental.pallas import tpu_sc as plsc

HIDDEN = 128
EPS = 1e-12
NC = 2
NS = 16
NW = NC * NS
CHUNK = 128
DEPTH = 4
BT = 512


def _sc_gather_body(ids_hbm, wt_hbm, out_hbm, idx_v, b0, b1, b2, b3,
                    g0, g1, g2, g3, o0, o1, o2, o3):
    bufs = (b0, b1, b2, b3)
    gsem = (g0, g1, g2, g3)
    osem = (o0, o1, o2, o3)
    wid = lax.axis_index("s") * NC + lax.axis_index("c")
    n = ids_hbm.shape[0] // NW
    base = wid * n
    pltpu.sync_copy(ids_hbm.at[pl.ds(base, n)], idx_v)

    def out_slice(c):
        return out_hbm.at[pl.ds((base + c) * CHUNK, CHUNK)]

    for b in range(DEPTH):
        pltpu.async_copy(wt_hbm.at[idx_v.at[b]], bufs[b], gsem[b])

    def grp_body(gi, carry):
        for b in range(DEPTH):
            c = gi * DEPTH + b
            pltpu.make_async_copy(
                wt_hbm.at[idx_v.at[c]], bufs[b], gsem[b]).wait()
            pltpu.async_copy(bufs[b], out_slice(c), osem[b])
            nc = c + DEPTH

            @pl.when(nc < n)
            def _():
                pltpu.make_async_copy(bufs[b], out_slice(c), osem[b]).wait()
                pltpu.async_copy(wt_hbm.at[idx_v.at[nc]], bufs[b], gsem[b])
        return carry

    lax.fori_loop(0, n // DEPTH, grp_body, None)

    for b in range(DEPTH):
        c = n - DEPTH + b
        pltpu.make_async_copy(bufs[b], out_slice(c), osem[b]).wait()


def _ln_body(x_ref, s_ref, st_ref, g_ref, bt_ref, o_ref):
    x = x_ref[...]
    s = s_ref[...]
    row0 = st_ref[0:1, :]
    row1 = st_ref[1:2, :]
    e = x + row0 + s * (row1 - row0)
    mean = jnp.mean(e, axis=-1, keepdims=True)
    d = e - mean
    var = jnp.mean(d * d, axis=-1, keepdims=True)
    o_ref[...] = d * lax.rsqrt(var + EPS) * g_ref[...] + bt_ref[...]


def kernel(code_ids, seg_ids, word_table, seg_table, ln_gamma, ln_beta):
    B, L = code_ids.shape
    T = B * L
    ids2d = code_ids.reshape(T // CHUNK, CHUNK).astype(jnp.int32)

    mesh = plsc.VectorSubcoreMesh(
        core_axis_name="c", subcore_axis_name="s",
        num_cores=NC, num_subcores=NS)
    n_per_w = (T // CHUNK) // NW
    gathered = pl.kernel(
        _sc_gather_body,
        out_type=jax.ShapeDtypeStruct((T, HIDDEN), jnp.float32),
        mesh=mesh,
        scratch_types=(
            [pltpu.VMEM((n_per_w, CHUNK), jnp.int32)]
            + [pltpu.VMEM((CHUNK, HIDDEN), jnp.float32)] * DEPTH
            + [pltpu.SemaphoreType.DMA] * (2 * DEPTH)
        ),
    )(ids2d, word_table)

    seg_f = seg_ids.reshape(T, 1).astype(jnp.float32)
    out = pl.pallas_call(
        _ln_body,
        grid=(T // BT,),
        in_specs=[
            pl.BlockSpec((BT, HIDDEN), lambda i: (i, 0)),
            pl.BlockSpec((BT, 1), lambda i: (i, 0)),
            pl.BlockSpec((2, HIDDEN), lambda i: (0, 0)),
            pl.BlockSpec((1, HIDDEN), lambda i: (0, 0)),
            pl.BlockSpec((1, HIDDEN), lambda i: (0, 0)),
        ],
        out_specs=pl.BlockSpec((BT, HIDDEN), lambda i: (i, 0)),
        out_shape=jax.ShapeDtypeStruct((T, HIDDEN), jnp.float32),
    )(gathered, seg_f, seg_table,
      ln_gamma.reshape(1, HIDDEN), ln_beta.reshape(1, HIDDEN))
    return out.reshape(B, L, HIDDEN)


# speedup vs baseline: 7.0225x; 1.7974x over previous
import jax
import jax.numpy as jnp
from jax import lax
from jax.experimental import pallas as pl
from jax.experimental.pallas import tpu as pltpu
from jax.experimental.pallas import tpu_sc as plsc

HIDDEN = 128
EPS = 1e-12
NC = 2
NS = 16
NW = NC * NS
CHUNK = 128
DEPTH = 4
BT = 4096


def _sc_gather_body(ids_hbm, wt_hbm, out_hbm, idx_v, b0, b1, b2, b3,
                    g0, g1, g2, g3, o0, o1, o2, o3):
    bufs = (b0, b1, b2, b3)
    gsem = (g0, g1, g2, g3)
    osem = (o0, o1, o2, o3)
    wid = lax.axis_index("s") * NC + lax.axis_index("c")
    n = ids_hbm.shape[0] // NW
    base = wid * n
    pltpu.sync_copy(ids_hbm.at[pl.ds(base, n)], idx_v)

    def out_slice(c):
        return out_hbm.at[pl.ds((base + c) * CHUNK, CHUNK)]

    for b in range(DEPTH):
        pltpu.async_copy(wt_hbm.at[idx_v.at[b]], bufs[b], gsem[b])

    def grp_body(gi, carry):
        for b in range(DEPTH):
            c = gi * DEPTH + b
            pltpu.make_async_copy(
                wt_hbm.at[idx_v.at[c]], bufs[b], gsem[b]).wait()
            pltpu.async_copy(bufs[b], out_slice(c), osem[b])
            nc = c + DEPTH

            @pl.when(nc < n)
            def _():
                pltpu.make_async_copy(bufs[b], out_slice(c), osem[b]).wait()
                pltpu.async_copy(wt_hbm.at[idx_v.at[nc]], bufs[b], gsem[b])
        return carry

    lax.fori_loop(0, n // DEPTH, grp_body, None)

    for b in range(DEPTH):
        c = n - DEPTH + b
        pltpu.make_async_copy(bufs[b], out_slice(c), osem[b]).wait()


def _ln_body(x_ref, s_ref, st_ref, g_ref, bt_ref, o_ref):
    x = x_ref[...]
    s = s_ref[...]
    row0 = st_ref[0:1, :]
    row1 = st_ref[1:2, :]
    e = x + row0 + s * (row1 - row0)
    mean = jnp.mean(e, axis=-1, keepdims=True)
    d = e - mean
    var = jnp.mean(d * d, axis=-1, keepdims=True)
    o_ref[...] = d * lax.rsqrt(var + EPS) * g_ref[...] + bt_ref[...]


def kernel(code_ids, seg_ids, word_table, seg_table, ln_gamma, ln_beta):
    B, L = code_ids.shape
    T = B * L
    ids2d = code_ids.reshape(T // CHUNK, CHUNK).astype(jnp.int32)

    mesh = plsc.VectorSubcoreMesh(
        core_axis_name="c", subcore_axis_name="s",
        num_cores=NC, num_subcores=NS)
    n_per_w = (T // CHUNK) // NW
    gathered = pl.kernel(
        _sc_gather_body,
        out_type=jax.ShapeDtypeStruct((T, HIDDEN), jnp.float32),
        mesh=mesh,
        scratch_types=(
            [pltpu.VMEM((n_per_w, CHUNK), jnp.int32)]
            + [pltpu.VMEM((CHUNK, HIDDEN), jnp.float32)] * DEPTH
            + [pltpu.SemaphoreType.DMA] * (2 * DEPTH)
        ),
    )(ids2d, word_table)

    seg_f = seg_ids.reshape(T, 1).astype(jnp.float32)
    out = pl.pallas_call(
        _ln_body,
        grid=(T // BT,),
        in_specs=[
            pl.BlockSpec((BT, HIDDEN), lambda i: (i, 0)),
            pl.BlockSpec((BT, 1), lambda i: (i, 0)),
            pl.BlockSpec((2, HIDDEN), lambda i: (0, 0)),
            pl.BlockSpec((1, HIDDEN), lambda i: (0, 0)),
            pl.BlockSpec((1, HIDDEN), lambda i: (0, 0)),
        ],
        out_specs=pl.BlockSpec((BT, HIDDEN), lambda i: (i, 0)),
        out_shape=jax.ShapeDtypeStruct((T, HIDDEN), jnp.float32),
    )(gathered, seg_f, seg_table,
      ln_gamma.reshape(1, HIDDEN), ln_beta.reshape(1, HIDDEN))
    return out.reshape(B, L, HIDDEN)
